# baseline (speedup 1.0000x reference)
import functools

import jax
import jax.numpy as jnp
from jax import lax
from jax.experimental import pallas as pl
from jax.experimental.pallas import tpu as pltpu
from jax.experimental.pallas import tpu_sc as plsc

D = 128
L = 16
NC, NS = 2, 16
NW = NC * NS
MARGIN_F = 1.0


def _approx_sqrt(x):
    xc = jnp.maximum(x, jnp.float32(1e-30))
    i = lax.bitcast_convert_type(xc, jnp.int32)
    y = lax.bitcast_convert_type(jnp.int32(0x5F3759DF) - (i >> 1), jnp.float32)
    for _ in range(3):
        y = y * (jnp.float32(1.5) - jnp.float32(0.5) * xc * y * y)
    return xc * y


@functools.cache
def _build(B, C):
    WPW = B // NW
    n_chunks = WPW // C
    NBUF = 6
    n_main = (n_chunks // NBUF) * NBUF
    n_tail = n_chunks - n_main

    f32 = jnp.float32
    i32 = jnp.int32
    mesh = plsc.VectorSubcoreMesh(core_axis_name="c", subcore_axis_name="s")

    @functools.partial(
        pl.kernel,
        out_type=(
            jax.ShapeDtypeStruct((B,), f32),
            jax.ShapeDtypeStruct((B,), f32),
            jax.ShapeDtypeStruct((B,), f32),
        ),
        mesh=mesh,
        compiler_params=pltpu.CompilerParams(needs_layout_passes=False),
        scratch_types=(
            [pltpu.VMEM((4 * WPW,), i32),
             pltpu.VMEM((WPW,), i32)]
            + [pltpu.VMEM((NBUF, 4 * C, D), f32)] * 2
            + [pltpu.VMEM((NBUF, C, D), f32)] * 2
            + [pltpu.VMEM((WPW,), f32)] * 3
            + [pltpu.SemaphoreType.DMA] * 6
        ),
    )
    def energy_kernel(oidx_hbm, lidx_hbm,
                      obj_re, obj_im, rel_re, rel_im,
                      loss_hbm, e_hbm, ce_hbm,
                      oiv, liv,
                      ore_b, oim_b, rre_b, rim_b,
                      loss_v, e_v, ce_v,
                      sem0, sem1, sem2, sem3, sem4, sem5):
        wid = lax.axis_index("s") * NC + lax.axis_index("c")
        base = wid * WPW
        sems = (sem0, sem1, sem2, sem3, sem4, sem5)

        pltpu.sync_copy(oidx_hbm.at[pl.ds(base * 4, 4 * WPW)], oiv)
        pltpu.sync_copy(lidx_hbm.at[pl.ds(base, WPW)], liv)

        def streams(g, p):
            o_sl = oiv.at[pl.ds(g * 4 * C, 4 * C)]
            l_sl = liv.at[pl.ds(g * C, C)]
            return (
                (obj_re.at[o_sl], ore_b.at[p]),
                (obj_im.at[o_sl], oim_b.at[p]),
                (rel_re.at[l_sl], rre_b.at[p]),
                (rel_im.at[l_sl], rim_b.at[p]),
            )

        def issue(g, p):
            for src, dst in streams(g, p):
                pltpu.async_copy(src, dst, sems[p])

        def drain(g, p):
            for src, dst in streams(g, p):
                pltpu.make_async_copy(src, dst, sems[p]).wait()

        lane = lax.iota(i32, L)
        zero = jnp.zeros((L,), f32)

        def compute(g, p):
            off = g * C
            for q in range(0, C, L):
                def elem_body(i, carry):
                    res_e, res_ce = carry
                    e = q + i
                    acc_e = zero
                    acc_ce = zero
                    for j in range(0, D, L):
                        sl = pl.ds(j, L)
                        hre = ore_b[p, e, sl]
                        him = oim_b[p, e, sl]
                        tre = ore_b[p, C + e, sl]
                        tim = oim_b[p, C + e, sl]
                        cre = ore_b[p, 2 * C + e, sl]
                        cim = oim_b[p, 2 * C + e, sl]
                        dre = ore_b[p, 3 * C + e, sl]
                        dim_ = oim_b[p, 3 * C + e, sl]
                        lre = rre_b[p, e, sl]
                        lim = rim_b[p, e, sl]
                        er = hre * lre - him * lim - tre
                        ei = hre * lim + him * lre - tim
                        acc_e = acc_e + er * er + ei * ei
                        cr = cre * lre - cim * lim - dre
                        ci = cre * lim + cim * lre - dim_
                        acc_ce = acc_ce + cr * cr + ci * ci
                    s_e = jnp.sum(acc_e)
                    s_ce = jnp.sum(acc_ce)
                    lmask = lane == i
                    res_e = jnp.where(lmask, s_e, res_e)
                    res_ce = jnp.where(lmask, s_ce, res_ce)
                    return res_e, res_ce

                res_e, res_ce = lax.fori_loop(0, L, elem_body, (zero, zero))
                ev = _approx_sqrt(res_e)
                cev = _approx_sqrt(res_ce)
                sl16 = pl.ds(off + q, L)
                e_v[sl16] = ev
                ce_v[sl16] = cev
                loss_v[sl16] = jnp.maximum(
                    ev - cev + jnp.float32(MARGIN_F), jnp.float32(0.0))

        for pr in range(NBUF - 1):
            issue(pr, pr)

        @pl.loop(0, n_main, step=NBUF)
        def _chunk_group(g0):
            for b in range(NBUF):
                g = g0 + b

                @pl.when(g + NBUF - 1 < n_chunks)
                def _():
                    issue(g + NBUF - 1, (b + NBUF - 1) % NBUF)

                drain(g, b)
                compute(g, b)

        for g in range(n_main, n_chunks):
            drain(g, g % NBUF)
            compute(g, g % NBUF)

        pltpu.sync_copy(loss_v, loss_hbm.at[pl.ds(base, WPW)])
        pltpu.sync_copy(e_v, e_hbm.at[pl.ds(base, WPW)])
        pltpu.sync_copy(ce_v, ce_hbm.at[pl.ds(base, WPW)])

    return energy_kernel


def kernel(correct, corrupted, obj_re, obj_im, rel_re, rel_im):
    correct = correct.astype(jnp.int32)
    corrupted = corrupted.astype(jnp.int32)
    B = correct.shape[0]
    C = 16
    WPW = B // NW
    n_chunks = WPW // C
    hctc = jnp.stack(
        [correct[:, 0], correct[:, 2], corrupted[:, 0], corrupted[:, 2]],
        axis=0,
    ).reshape(4, NW, n_chunks, C)
    obj_idx = jnp.transpose(hctc, (1, 2, 0, 3)).reshape(-1)
    l_idx = correct[:, 1]
    fn = _build(B, C)
    return fn(obj_idx, l_idx, obj_re, obj_im, rel_re, rel_im)


# speedup vs baseline: 1.0126x; 1.0126x over previous
import functools

import jax
import jax.numpy as jnp
from jax import lax
from jax.experimental import pallas as pl
from jax.experimental.pallas import tpu as pltpu
from jax.experimental.pallas import tpu_sc as plsc

D = 128
L = 16
NC, NS = 2, 16
NW = NC * NS
MARGIN_F = 1.0


def _approx_sqrt(x):
    xc = jnp.maximum(x, jnp.float32(1e-30))
    i = lax.bitcast_convert_type(xc, jnp.int32)
    y = lax.bitcast_convert_type(jnp.int32(0x5F3759DF) - (i >> 1), jnp.float32)
    for _ in range(3):
        y = y * (jnp.float32(1.5) - jnp.float32(0.5) * xc * y * y)
    return xc * y


@functools.cache
def _build(B, C):
    WPW = B // NW
    n_chunks = WPW // C
    NBUF = 3
    n_main = (n_chunks // NBUF) * NBUF

    f32 = jnp.float32
    i32 = jnp.int32
    mesh = plsc.VectorSubcoreMesh(core_axis_name="c", subcore_axis_name="s")

    @functools.partial(
        pl.kernel,
        out_type=(
            jax.ShapeDtypeStruct((B,), f32),
            jax.ShapeDtypeStruct((B,), f32),
            jax.ShapeDtypeStruct((B,), f32),
        ),
        mesh=mesh,
        compiler_params=pltpu.CompilerParams(needs_layout_passes=False),
        scratch_types=(
            [pltpu.VMEM((4 * WPW,), i32),
             pltpu.VMEM((WPW,), i32)]
            + [pltpu.VMEM((NBUF, 4 * C, D), f32)] * 2
            + [pltpu.VMEM((NBUF, C, D), f32)] * 2
            + [pltpu.VMEM((WPW,), f32)] * 3
            + [pltpu.SemaphoreType.DMA] * 3
        ),
    )
    def energy_kernel(oidx_hbm, lidx_hbm,
                      obj_re, obj_im, rel_re, rel_im,
                      loss_hbm, e_hbm, ce_hbm,
                      oiv, liv,
                      ore_b, oim_b, rre_b, rim_b,
                      loss_v, e_v, ce_v,
                      sem0, sem1, sem2):
        wid = lax.axis_index("s") * NC + lax.axis_index("c")
        base = wid * WPW
        sems = (sem0, sem1, sem2)

        pltpu.sync_copy(oidx_hbm.at[pl.ds(base * 4, 4 * WPW)], oiv)
        pltpu.sync_copy(lidx_hbm.at[pl.ds(base, WPW)], liv)

        def streams(g, p):
            o_sl = oiv.at[pl.ds(g * 4 * C, 4 * C)]
            l_sl = liv.at[pl.ds(g * C, C)]
            return (
                (obj_re.at[o_sl], ore_b.at[p]),
                (obj_im.at[o_sl], oim_b.at[p]),
                (rel_re.at[l_sl], rre_b.at[p]),
                (rel_im.at[l_sl], rim_b.at[p]),
            )

        def issue(g, p):
            for src, dst in streams(g, p):
                pltpu.async_copy(src, dst, sems[p])

        def drain(g, p):
            for src, dst in streams(g, p):
                pltpu.make_async_copy(src, dst, sems[p]).wait()

        lane = lax.iota(i32, L)
        zero = jnp.zeros((L,), f32)

        def compute(g, p):
            off = g * C
            for q in range(0, C, L):
                def elem_body(i, carry):
                    res_e, res_ce = carry
                    e = q + i
                    acc_e = zero
                    acc_ce = zero
                    for j in range(0, D, L):
                        sl = pl.ds(j, L)
                        hre = ore_b[p, e, sl]
                        him = oim_b[p, e, sl]
                        tre = ore_b[p, C + e, sl]
                        tim = oim_b[p, C + e, sl]
                        cre = ore_b[p, 2 * C + e, sl]
                        cim = oim_b[p, 2 * C + e, sl]
                        dre = ore_b[p, 3 * C + e, sl]
                        dim_ = oim_b[p, 3 * C + e, sl]
                        lre = rre_b[p, e, sl]
                        lim = rim_b[p, e, sl]
                        er = hre * lre - him * lim - tre
                        ei = hre * lim + him * lre - tim
                        acc_e = acc_e + er * er + ei * ei
                        cr = cre * lre - cim * lim - dre
                        ci = cre * lim + cim * lre - dim_
                        acc_ce = acc_ce + cr * cr + ci * ci
                    s_e = jnp.sum(acc_e)
                    s_ce = jnp.sum(acc_ce)
                    lmask = lane == i
                    res_e = jnp.where(lmask, s_e, res_e)
                    res_ce = jnp.where(lmask, s_ce, res_ce)
                    return res_e, res_ce

                res_e, res_ce = lax.fori_loop(0, L, elem_body, (zero, zero))
                ev = _approx_sqrt(res_e)
                cev = _approx_sqrt(res_ce)
                sl16 = pl.ds(off + q, L)
                e_v[sl16] = ev
                ce_v[sl16] = cev
                loss_v[sl16] = jnp.maximum(
                    ev - cev + jnp.float32(MARGIN_F), jnp.float32(0.0))

        for pr in range(NBUF - 1):
            issue(pr, pr)

        @pl.loop(0, n_main, step=NBUF)
        def _chunk_group(g0):
            for b in range(NBUF):
                g = g0 + b

                @pl.when(g + NBUF - 1 < n_chunks)
                def _():
                    issue(g + NBUF - 1, (b + NBUF - 1) % NBUF)

                drain(g, b)
                compute(g, b)

        for g in range(n_main, n_chunks):
            drain(g, g % NBUF)
            compute(g, g % NBUF)

        pltpu.sync_copy(loss_v, loss_hbm.at[pl.ds(base, WPW)])
        pltpu.sync_copy(e_v, e_hbm.at[pl.ds(base, WPW)])
        pltpu.sync_copy(ce_v, ce_hbm.at[pl.ds(base, WPW)])

    return energy_kernel


def kernel(correct, corrupted, obj_re, obj_im, rel_re, rel_im):
    correct = correct.astype(jnp.int32)
    corrupted = corrupted.astype(jnp.int32)
    B = correct.shape[0]
    C = 32
    WPW = B // NW
    n_chunks = WPW // C
    hctc = jnp.stack(
        [correct[:, 0], correct[:, 2], corrupted[:, 0], corrupted[:, 2]],
        axis=0,
    ).reshape(4, NW, n_chunks, C)
    obj_idx = jnp.transpose(hctc, (1, 2, 0, 3)).reshape(-1)
    l_idx = correct[:, 1]
    fn = _build(B, C)
    return fn(obj_idx, l_idx, obj_re, obj_im, rel_re, rel_im)


# speedup vs baseline: 1.0479x; 1.0349x over previous
import functools

import jax
import jax.numpy as jnp
from jax import lax
from jax.experimental import pallas as pl
from jax.experimental.pallas import tpu as pltpu
from jax.experimental.pallas import tpu_sc as plsc

D = 128
L = 16
NC, NS = 2, 16
NW = NC * NS
MARGIN_F = 1.0


def _approx_sqrt(x):
    xc = jnp.maximum(x, jnp.float32(1e-30))
    i = lax.bitcast_convert_type(xc, jnp.int32)
    y = lax.bitcast_convert_type(jnp.int32(0x5F3759DF) - (i >> 1), jnp.float32)
    for _ in range(3):
        y = y * (jnp.float32(1.5) - jnp.float32(0.5) * xc * y * y)
    return xc * y


@functools.cache
def _build(B, C):
    WPW = B // NW
    n_chunks = WPW // C
    NBUF = 4
    assert n_chunks % NBUF == 0

    f32 = jnp.float32
    i32 = jnp.int32
    mesh = plsc.VectorSubcoreMesh(core_axis_name="c", subcore_axis_name="s")

    @functools.partial(
        pl.kernel,
        out_type=(
            jax.ShapeDtypeStruct((B,), f32),
            jax.ShapeDtypeStruct((B,), f32),
            jax.ShapeDtypeStruct((B,), f32),
        ),
        mesh=mesh,
        compiler_params=pltpu.CompilerParams(needs_layout_passes=False),
        scratch_types=(
            [pltpu.VMEM((4 * WPW,), i32),
             pltpu.VMEM((WPW,), i32)]
            + [pltpu.VMEM((NBUF, 4 * C, D), f32)] * 2
            + [pltpu.VMEM((NBUF, C, D), f32)] * 2
            + [pltpu.VMEM((WPW,), f32)] * 3
            + [pltpu.SemaphoreType.DMA] * 4
        ),
    )
    def energy_kernel(oidx_hbm, lidx_hbm,
                      obj_re, obj_im, rel_re, rel_im,
                      loss_hbm, e_hbm, ce_hbm,
                      oiv, liv,
                      ore_b, oim_b, rre_b, rim_b,
                      loss_v, e_v, ce_v,
                      sem0, sem1, sem2, sem3):
        wid = lax.axis_index("s") * NC + lax.axis_index("c")
        base = wid * WPW
        sems = (sem0, sem1, sem2, sem3)

        pltpu.sync_copy(oidx_hbm.at[pl.ds(base * 4, 4 * WPW)], oiv)
        pltpu.sync_copy(lidx_hbm.at[pl.ds(base, WPW)], liv)

        def streams(g, p):
            o_sl0 = oiv.at[pl.ds(g * 4 * C, 2 * C)]
            o_sl1 = oiv.at[pl.ds(g * 4 * C + 2 * C, 2 * C)]
            l_sl = liv.at[pl.ds(g * C, C)]
            return (
                (obj_re.at[o_sl0], ore_b.at[p, pl.ds(0, 2 * C)]),
                (obj_im.at[o_sl0], oim_b.at[p, pl.ds(0, 2 * C)]),
                (obj_re.at[o_sl1], ore_b.at[p, pl.ds(2 * C, 2 * C)]),
                (obj_im.at[o_sl1], oim_b.at[p, pl.ds(2 * C, 2 * C)]),
                (rel_re.at[l_sl], rre_b.at[p]),
                (rel_im.at[l_sl], rim_b.at[p]),
            )

        def issue(g, p):
            for src, dst in streams(g, p):
                pltpu.async_copy(src, dst, sems[p])

        def drain(g, p):
            for src, dst in streams(g, p):
                pltpu.make_async_copy(src, dst, sems[p]).wait()

        lane = lax.iota(i32, L)
        zero = jnp.zeros((L,), f32)

        def compute(g, p):
            off = g * C
            for q in range(0, C, L):
                def elem_body(i, carry):
                    res_e, res_ce = carry
                    e = q + i
                    acc_e = zero
                    acc_ce = zero
                    for j in range(0, D, L):
                        sl = pl.ds(j, L)
                        hre = ore_b[p, e, sl]
                        him = oim_b[p, e, sl]
                        tre = ore_b[p, C + e, sl]
                        tim = oim_b[p, C + e, sl]
                        cre = ore_b[p, 2 * C + e, sl]
                        cim = oim_b[p, 2 * C + e, sl]
                        dre = ore_b[p, 3 * C + e, sl]
                        dim_ = oim_b[p, 3 * C + e, sl]
                        lre = rre_b[p, e, sl]
                        lim = rim_b[p, e, sl]
                        er = hre * lre - him * lim - tre
                        ei = hre * lim + him * lre - tim
                        acc_e = acc_e + er * er + ei * ei
                        cr = cre * lre - cim * lim - dre
                        ci = cre * lim + cim * lre - dim_
                        acc_ce = acc_ce + cr * cr + ci * ci
                    s_e = jnp.sum(acc_e)
                    s_ce = jnp.sum(acc_ce)
                    lmask = lane == i
                    res_e = jnp.where(lmask, s_e, res_e)
                    res_ce = jnp.where(lmask, s_ce, res_ce)
                    return res_e, res_ce

                res_e, res_ce = lax.fori_loop(0, L, elem_body, (zero, zero))
                ev = _approx_sqrt(res_e)
                cev = _approx_sqrt(res_ce)
                sl16 = pl.ds(off + q, L)
                e_v[sl16] = ev
                ce_v[sl16] = cev
                loss_v[sl16] = jnp.maximum(
                    ev - cev + jnp.float32(MARGIN_F), jnp.float32(0.0))

        for pr in range(NBUF - 1):
            issue(pr, pr)

        @pl.loop(0, n_chunks, step=NBUF)
        def _chunk_group(g0):
            for b in range(NBUF):
                g = g0 + b

                @pl.when(g + NBUF - 1 < n_chunks)
                def _():
                    issue(g + NBUF - 1, (b + NBUF - 1) % NBUF)

                drain(g, b)
                compute(g, b)

        pltpu.sync_copy(loss_v, loss_hbm.at[pl.ds(base, WPW)])
        pltpu.sync_copy(e_v, e_hbm.at[pl.ds(base, WPW)])
        pltpu.sync_copy(ce_v, ce_hbm.at[pl.ds(base, WPW)])

    return energy_kernel


def kernel(correct, corrupted, obj_re, obj_im, rel_re, rel_im):
    correct = correct.astype(jnp.int32)
    corrupted = corrupted.astype(jnp.int32)
    B = correct.shape[0]
    C = 16
    WPW = B // NW
    n_chunks = WPW // C
    hctc = jnp.stack(
        [correct[:, 0], correct[:, 2], corrupted[:, 0], corrupted[:, 2]],
        axis=0,
    ).reshape(4, NW, n_chunks, C)
    obj_idx = jnp.transpose(hctc, (1, 2, 0, 3)).reshape(-1)
    l_idx = correct[:, 1]
    fn = _build(B, C)
    return fn(obj_idx, l_idx, obj_re, obj_im, rel_re, rel_im)


# speedup vs baseline: 1.0541x; 1.0059x over previous
import functools

import jax
import jax.numpy as jnp
from jax import lax
from jax.experimental import pallas as pl
from jax.experimental.pallas import tpu as pltpu
from jax.experimental.pallas import tpu_sc as plsc

D = 128
L = 16
NC, NS = 2, 16
NW = NC * NS
MARGIN_F = 1.0


def _approx_sqrt(x):
    xc = jnp.maximum(x, jnp.float32(1e-30))
    i = lax.bitcast_convert_type(xc, jnp.int32)
    y = lax.bitcast_convert_type(jnp.int32(0x5F3759DF) - (i >> 1), jnp.float32)
    for _ in range(3):
        y = y * (jnp.float32(1.5) - jnp.float32(0.5) * xc * y * y)
    return xc * y


@functools.cache
def _build(B, C):
    WPW = B // NW
    n_chunks = WPW // C
    NBUF = 4
    assert n_chunks % NBUF == 0

    f32 = jnp.float32
    i32 = jnp.int32
    mesh = plsc.VectorSubcoreMesh(core_axis_name="c", subcore_axis_name="s")

    @functools.partial(
        pl.kernel,
        out_type=(
            jax.ShapeDtypeStruct((B,), f32),
            jax.ShapeDtypeStruct((B,), f32),
            jax.ShapeDtypeStruct((B,), f32),
        ),
        mesh=mesh,
        compiler_params=pltpu.CompilerParams(needs_layout_passes=False),
        scratch_types=(
            [pltpu.VMEM((4 * WPW,), i32),
             pltpu.VMEM((WPW,), i32)]
            + [pltpu.VMEM((NBUF, 4 * C, D), f32)] * 2
            + [pltpu.VMEM((NBUF, C, D), f32)] * 2
            + [pltpu.VMEM((WPW,), f32)] * 3
            + [pltpu.SemaphoreType.DMA] * 4
        ),
    )
    def energy_kernel(oidx_hbm, lidx_hbm,
                      obj_re, obj_im, rel_re, rel_im,
                      loss_hbm, e_hbm, ce_hbm,
                      oiv, liv,
                      ore_b, oim_b, rre_b, rim_b,
                      loss_v, e_v, ce_v,
                      sem0, sem1, sem2, sem3):
        wid = lax.axis_index("s") * NC + lax.axis_index("c")
        base = wid * WPW
        sems = (sem0, sem1, sem2, sem3)

        pltpu.sync_copy(oidx_hbm.at[pl.ds(base * 4, 4 * WPW)], oiv)
        pltpu.sync_copy(lidx_hbm.at[pl.ds(base, WPW)], liv)

        def streams(g, p):
            o_sl = oiv.at[pl.ds(g * 4 * C, 4 * C)]
            l_sl = liv.at[pl.ds(g * C, C)]
            return (
                (obj_re.at[o_sl], ore_b.at[p]),
                (obj_im.at[o_sl], oim_b.at[p]),
                (rel_re.at[l_sl], rre_b.at[p]),
                (rel_im.at[l_sl], rim_b.at[p]),
            )

        def issue(g, p):
            for src, dst in streams(g, p):
                pltpu.async_copy(src, dst, sems[p])

        def drain(g, p):
            for src, dst in streams(g, p):
                pltpu.make_async_copy(src, dst, sems[p]).wait()

        lane = lax.iota(i32, L)
        zero = jnp.zeros((L,), f32)

        def compute(g, p):
            off = g * C
            for q in range(0, C, L):
                def elem_body(i, carry):
                    res_e, res_ce = carry
                    e = q + i
                    acc_e = zero
                    acc_ce = zero
                    for j in range(0, D, L):
                        sl = pl.ds(j, L)
                        hre = ore_b[p, e, sl]
                        him = oim_b[p, e, sl]
                        tre = ore_b[p, C + e, sl]
                        tim = oim_b[p, C + e, sl]
                        cre = ore_b[p, 2 * C + e, sl]
                        cim = oim_b[p, 2 * C + e, sl]
                        dre = ore_b[p, 3 * C + e, sl]
                        dim_ = oim_b[p, 3 * C + e, sl]
                        lre = rre_b[p, e, sl]
                        lim = rim_b[p, e, sl]
                        er = hre * lre - him * lim - tre
                        ei = hre * lim + him * lre - tim
                        acc_e = acc_e + er * er + ei * ei
                        cr = cre * lre - cim * lim - dre
                        ci = cre * lim + cim * lre - dim_
                        acc_ce = acc_ce + cr * cr + ci * ci
                    s_e = jnp.sum(acc_e)
                    s_ce = jnp.sum(acc_ce)
                    lmask = lane == i
                    res_e = jnp.where(lmask, s_e, res_e)
                    res_ce = jnp.where(lmask, s_ce, res_ce)
                    return res_e, res_ce

                res_e, res_ce = lax.fori_loop(0, L, elem_body, (zero, zero))
                ev = _approx_sqrt(res_e)
                cev = _approx_sqrt(res_ce)
                sl16 = pl.ds(off + q, L)
                e_v[sl16] = ev
                ce_v[sl16] = cev
                loss_v[sl16] = jnp.maximum(
                    ev - cev + jnp.float32(MARGIN_F), jnp.float32(0.0))

        for pr in range(NBUF - 1):
            issue(pr, pr)

        @pl.loop(0, n_chunks, step=NBUF)
        def _chunk_group(g0):
            for b in range(NBUF):
                g = g0 + b

                @pl.when(g + NBUF - 1 < n_chunks)
                def _():
                    issue(g + NBUF - 1, (b + NBUF - 1) % NBUF)

                drain(g, b)
                compute(g, b)

        pltpu.sync_copy(loss_v, loss_hbm.at[pl.ds(base, WPW)])
        pltpu.sync_copy(e_v, e_hbm.at[pl.ds(base, WPW)])
        pltpu.sync_copy(ce_v, ce_hbm.at[pl.ds(base, WPW)])

    return energy_kernel


def kernel(correct, corrupted, obj_re, obj_im, rel_re, rel_im):
    correct = correct.astype(jnp.int32)
    corrupted = corrupted.astype(jnp.int32)
    B = correct.shape[0]
    C = 16
    WPW = B // NW
    n_chunks = WPW // C
    hctc = jnp.stack(
        [correct[:, 0], correct[:, 2], corrupted[:, 0], corrupted[:, 2]],
        axis=0,
    ).reshape(4, NW, n_chunks, C)
    obj_idx = jnp.transpose(hctc, (1, 2, 0, 3)).reshape(-1)
    l_idx = correct[:, 1]
    fn = _build(B, C)
    return fn(obj_idx, l_idx, obj_re, obj_im, rel_re, rel_im)
